# baseline (speedup 1.0000x reference)
import functools

import jax
import jax.numpy as jnp
from jax import lax
from jax.experimental import pallas as pl
from jax.experimental.pallas import tpu as pltpu
from jax.experimental.pallas import tpu_sc as plsc

_CHUNK = 128


def _emb_kernel(table_hbm, idx_hbm, out_hbm, idx_v, rows_v, sem, *,
                num_cores, chunks_per_worker):
    wid = lax.axis_index("s") * num_cores + lax.axis_index("c")
    base = wid * chunks_per_worker
    pltpu.sync_copy(idx_hbm.at[pl.ds(base, chunks_per_worker)], idx_v)
    copies = [
        pltpu.async_copy(table_hbm.at[idx_v.at[j]], rows_v.at[j], sem)
        for j in range(chunks_per_worker)
    ]
    for c in copies:
        c.wait()
    pltpu.sync_copy(rows_v, out_hbm.at[pl.ds(base, chunks_per_worker)])


def kernel(labels, embedding_table):
    (batch,) = labels.shape
    _, hidden = embedding_table.shape
    info = plsc.get_sparse_core_info()
    num_workers = info.num_cores * info.num_subcores
    chunks = batch // _CHUNK
    chunks_per_worker = chunks // num_workers

    idx2d = labels.astype(jnp.int32).reshape(chunks, _CHUNK)
    mesh = plsc.VectorSubcoreMesh(core_axis_name="c", subcore_axis_name="s")

    emb = pl.kernel(
        functools.partial(
            _emb_kernel,
            num_cores=info.num_cores,
            chunks_per_worker=chunks_per_worker,
        ),
        out_type=jax.ShapeDtypeStruct((chunks, _CHUNK, hidden), jnp.float32),
        mesh=mesh,
        scratch_types=[
            pltpu.VMEM((chunks_per_worker, _CHUNK), jnp.int32),
            pltpu.VMEM((chunks_per_worker, _CHUNK, hidden), jnp.float32),
            pltpu.SemaphoreType.DMA,
        ],
        compiler_params=pltpu.CompilerParams(use_tc_tiling_on_sc=False),
    )
    out = emb(embedding_table, idx2d)
    return out.reshape(batch, hidden)
